# baseline (speedup 1.0000x reference)
import jax
import jax.numpy as jnp
from jax import lax
from jax.experimental import pallas as pl
from jax.experimental.pallas import tpu as pltpu
from jax.experimental.pallas import tpu_sc as plsc

_NC = 2
_NS = 16
_SR = 400

_NB = 4
_SG = 8
_NI = 4


def _split(total, parts, s):
    base, rem = divmod(total, parts)
    start = base * s + jnp.minimum(s, rem)
    cnt = jnp.where(s < rem, base + 1, base)
    return start, cnt


def _make_sc_hop(T2, A, CH, H, core_stride, pass_stride, acc_split, out_rows):
    CNT = CH // _NS
    NSG = CNT // _SG
    NBODY = NSG // _NI
    assert CNT % (_SG * _NI) == 0
    mesh = plsc.VectorSubcoreMesh(core_axis_name="c", subcore_axis_name="s")

    def body(gs0_hbm, gs1_hbm, tbl_hbm, z_hbm, out_hbm, acc_sh, tbl_sh,
             ibs, rows, isem, gsem, ssem):
        c = lax.axis_index("c")
        s = lax.axis_index("s")

        za_start, za_cnt = _split(A // _SR, _NS, s)
        tb_start, tb_cnt = _split(T2 // _SR, _NS, s)

        def zero_acc():
            def zchunk(k, carry):
                pltpu.sync_copy(z_hbm, acc_sh.at[pl.ds(k * _SR, _SR)])
                return carry

            lax.fori_loop(za_start, za_start + za_cnt, zchunk, 0)

        def load_table(p):
            base = c * core_stride + p * pass_stride

            def tchunk(k, carry):
                pltpu.sync_copy(tbl_hbm.at[pl.ds(base + k * _SR, _SR)],
                                tbl_sh.at[pl.ds(k * _SR, _SR)])
                return carry

            lax.fori_loop(tb_start, tb_start + tb_cnt, tchunk, 0)

        def out_copy(row0):
            def ochunk(k, carry):
                pltpu.sync_copy(acc_sh.at[pl.ds(k * _SR, _SR)],
                                out_hbm.at[c, pl.ds(row0 + k * _SR, _SR)])
                return carry

            lax.fori_loop(za_start, za_start + za_cnt, ochunk, 0)

        def main(gs_hbm):
            def iload(sg, p):
                base = s * CNT + sg * _SG
                pltpu.make_async_copy(gs_hbm.at[pl.ds(base, _SG)], ibs[p],
                                      isem[p]).start()

            def iwait(p):
                pltpu.make_async_copy(gs_hbm.at[pl.ds(0, _SG)], ibs[p],
                                      isem[p]).wait()

            def gath(p, r, b):
                pltpu.make_async_copy(tbl_sh.at[ibs[p].at[r, 0]], rows[b],
                                      gsem[b]).start()

            def gwait(b):
                pltpu.make_async_copy(tbl_sh.at[ibs[0].at[0, 0]], rows[b],
                                      gsem[b]).wait()

            def scat(p, r, b):
                pltpu.make_async_copy(rows[b], acc_sh.at[ibs[p].at[r, 1]],
                                      ssem[b]).start(add=True)

            def swait(b):
                pltpu.make_async_copy(rows[0], acc_sh.at[ibs[0].at[0, 1]],
                                      ssem[b]).wait()

            iload(0, 0)
            iwait(0)
            for b in range(_NB):
                gath(0, b, b)

            def run_sg(g, u):
                p = u
                pn = (u + 1) % _NI
                sg = g * _NI + u
                last = (u == _NI - 1)

                def guarded(fn):
                    if last:
                        @pl.when(g < NBODY - 1)
                        def _():
                            fn()
                    else:
                        fn()

                guarded(lambda: iload(sg + 1, pn))
                for k in range(_SG):
                    b = k % _NB
                    gwait(b)
                    scat(p, k, b)
                    if k == _NB:
                        guarded(lambda: iwait(pn))
                    if k < _SG - _NB:
                        swait(b)
                        gath(p, k + _NB, b)
                    else:
                        kk = k - (_SG - _NB)
                        guarded(lambda bb=b, kk=kk: (swait(bb),
                                                     gath(pn, kk, bb)))

            def group(g, carry):
                for u in range(_NI):
                    run_sg(g, u)
                return carry

            lax.fori_loop(0, NBODY, group, 0)
            for b in range(_NB):
                swait(b)

        load_table(0)
        zero_acc()
        plsc.subcore_barrier()
        main(gs0_hbm)
        plsc.subcore_barrier()
        if acc_split:
            out_copy(0)
            zero_acc()
        if pass_stride:
            load_table(1)
        plsc.subcore_barrier()
        main(gs1_hbm)
        plsc.subcore_barrier()
        out_copy(A if acc_split else 0)

    return pl.kernel(
        body,
        out_type=jax.ShapeDtypeStruct((_NC, out_rows, H), jnp.float32),
        mesh=mesh,
        compiler_params=pltpu.CompilerParams(use_tc_tiling_on_sc=False),
        scratch_types=[
            pltpu.VMEM_SHARED((A + 8, H), jnp.float32),
            pltpu.VMEM_SHARED((T2, H), jnp.float32),
            [pltpu.VMEM((_SG, 2, 128), jnp.int32)] * _NI,
            [pltpu.VMEM((128, H), jnp.float32)] * _NB,
            [pltpu.SemaphoreType.DMA] * _NI,
            [pltpu.SemaphoreType.DMA] * _NB,
            [pltpu.SemaphoreType.DMA] * _NB,
        ],
    )


def _make_tc_scale(M, H, BN):
    def body(x, deg, out):
        out[...] = x[...] * deg[...]

    return pl.pallas_call(
        body,
        grid=(M // BN,),
        in_specs=[
            pl.BlockSpec((_NC, BN, H), lambda i: (0, i, 0)),
            pl.BlockSpec((1, BN, 1), lambda i: (0, i, 0)),
        ],
        out_specs=pl.BlockSpec((_NC, BN, H), lambda i: (0, i, 0)),
        out_shape=jax.ShapeDtypeStruct((_NC, M, H), jnp.float32),
    )


def _make_tc_final(N, D, BN):
    def body(xv2, x0, wt, degv, ab, out):
        a = ab[0, 0]
        b = ab[0, 1]
        xv = jnp.concatenate([xv2[0], xv2[1]], axis=1)
        xi = (1.0 - a) * (xv * degv[...]) + a * x0[...]
        mm = jnp.dot(xi, wt[...], preferred_element_type=jnp.float32)
        out[...] = (1.0 - b) * xi + b * mm

    H = D // 2
    return pl.pallas_call(
        body,
        grid=(N // BN,),
        in_specs=[
            pl.BlockSpec((_NC, BN, H), lambda i: (0, i, 0)),
            pl.BlockSpec((BN, D), lambda i: (i, 0)),
            pl.BlockSpec((D, D), lambda i: (0, 0)),
            pl.BlockSpec((BN, 1), lambda i: (i, 0)),
            pl.BlockSpec(memory_space=pltpu.SMEM),
        ],
        out_specs=pl.BlockSpec((BN, D), lambda i: (i, 0)),
        out_shape=jax.ShapeDtypeStruct((N, D), jnp.float32),
    )


def kernel(X, vertex, edges, X0, alpha, beta, W, degE, degV):
    N, D = X.shape
    E = vertex.shape[0]
    M = degE.shape[0]
    H = D // 2
    MH = M // 2

    xr = jnp.concatenate([X[:, :H], X[:, H:]], axis=0)
    zeros = jnp.zeros((_SR, H), jnp.float32)

    CH = -(-E // (128 * _NS * _SG * _NI)) * _NS * _SG * _NI
    EP = CH * 128

    def pad(a, val):
        return jnp.concatenate([a, jnp.full((EP - E,), val, jnp.int32)])

    def pack(g, sc):
        return jnp.stack([g.reshape(CH, 128), sc.reshape(CH, 128)], axis=1)

    vp = pad(vertex, 0)
    ep = pad(edges, M)

    h1 = [pack(vp, jnp.where((ep >= p * MH) & (ep < (p + 1) * MH),
                             ep - p * MH, MH)) for p in range(2)]
    xe2 = _make_sc_hop(N, MH, CH, H, N, 0, True, M)(h1[0], h1[1], xr, zeros)

    xe2 = _make_tc_scale(M, H, 1000)(xe2, degE.reshape(1, M, 1))

    h2 = []
    for p in range(2):
        ins = (ep >= p * MH) & (ep < (p + 1) * MH)
        h2.append(pack(jnp.where(ins, ep - p * MH, 0),
                       jnp.where(ins, vp, N)))
    xv2 = _make_sc_hop(MH, N, CH, H, M, MH, False, N)(
        h2[0], h2[1], xe2.reshape(2 * M, H), zeros)

    ab = jnp.stack([jnp.float32(alpha), jnp.float32(beta)]).reshape(1, 2)
    return _make_tc_final(N, D, 1000)(xv2, X0, W.T, degV, ab)


# speedup vs baseline: 1.3481x; 1.3481x over previous
import jax
import jax.numpy as jnp
from jax import lax
from jax.experimental import pallas as pl
from jax.experimental.pallas import tpu as pltpu
from jax.experimental.pallas import tpu_sc as plsc

_NC = 2
_NS = 16
_SR = 400

_NB = 8
_SG = 16
_NI = 4
_CK = 64


def _split(total, parts, s):
    base, rem = divmod(total, parts)
    start = base * s + jnp.minimum(s, rem)
    cnt = jnp.where(s < rem, base + 1, base)
    return start, cnt


def _make_sc_hop(T2, A, CH, H, core_stride, pass_stride, acc_split, out_rows):
    CNT = CH // _NS
    NSG = CNT // _SG
    NBODY = NSG // _NI
    assert CNT % (_SG * _NI) == 0
    mesh = plsc.VectorSubcoreMesh(core_axis_name="c", subcore_axis_name="s")

    def body(gs0_hbm, gs1_hbm, tbl_hbm, z_hbm, out_hbm, acc_sh, tbl_sh,
             ibs, rows, isem, gsem, ssem):
        c = lax.axis_index("c")
        s = lax.axis_index("s")

        za_start, za_cnt = _split(A // _SR, _NS, s)
        tb_start, tb_cnt = _split(T2 // _SR, _NS, s)

        def zero_acc():
            def zchunk(k, carry):
                pltpu.sync_copy(z_hbm, acc_sh.at[pl.ds(k * _SR, _SR)])
                return carry

            lax.fori_loop(za_start, za_start + za_cnt, zchunk, 0)

        def load_table(p):
            base = c * core_stride + p * pass_stride

            def tchunk(k, carry):
                pltpu.sync_copy(tbl_hbm.at[pl.ds(base + k * _SR, _SR)],
                                tbl_sh.at[pl.ds(k * _SR, _SR)])
                return carry

            lax.fori_loop(tb_start, tb_start + tb_cnt, tchunk, 0)

        def out_copy(row0):
            def ochunk(k, carry):
                pltpu.sync_copy(acc_sh.at[pl.ds(k * _SR, _SR)],
                                out_hbm.at[c, pl.ds(row0 + k * _SR, _SR)])
                return carry

            lax.fori_loop(za_start, za_start + za_cnt, ochunk, 0)

        def main(gs_hbm):
            def iload(sg, p):
                base = s * CNT + sg * _SG
                pltpu.make_async_copy(gs_hbm.at[pl.ds(base, _SG)], ibs[p],
                                      isem[p]).start()

            def iwait(p):
                pltpu.make_async_copy(gs_hbm.at[pl.ds(0, _SG)], ibs[p],
                                      isem[p]).wait()

            def gath(p, r, b):
                pltpu.make_async_copy(tbl_sh.at[ibs[p].at[r, 0]], rows[b],
                                      gsem[b]).start()

            def gwait(b):
                pltpu.make_async_copy(tbl_sh.at[ibs[0].at[0, 0]], rows[b],
                                      gsem[b]).wait()

            def scat(p, r, b):
                pltpu.make_async_copy(rows[b], acc_sh.at[ibs[p].at[r, 1]],
                                      ssem[b]).start(add=True)

            def swait(b):
                pltpu.make_async_copy(rows[0], acc_sh.at[ibs[0].at[0, 1]],
                                      ssem[b]).wait()

            iload(0, 0)
            iwait(0)
            for b in range(_NB):
                gath(0, b, b)

            def run_sg(g, u):
                p = u
                pn = (u + 1) % _NI
                sg = g * _NI + u
                last = (u == _NI - 1)

                def guarded(fn):
                    if last:
                        @pl.when(g < NBODY - 1)
                        def _():
                            fn()
                    else:
                        fn()

                guarded(lambda: iload(sg + 1, pn))
                for k in range(_SG):
                    b = k % _NB
                    gwait(b)
                    scat(p, k, b)
                    if k == _NB:
                        guarded(lambda: iwait(pn))
                    if k < _SG - _NB:
                        swait(b)
                        gath(p, k + _NB, b)
                    else:
                        kk = k - (_SG - _NB)
                        guarded(lambda bb=b, kk=kk: (swait(bb),
                                                     gath(pn, kk, bb)))

            def group(g, carry):
                for u in range(_NI):
                    run_sg(g, u)
                return carry

            lax.fori_loop(0, NBODY, group, 0)
            for b in range(_NB):
                swait(b)

        load_table(0)
        zero_acc()
        plsc.subcore_barrier()
        main(gs0_hbm)
        plsc.subcore_barrier()
        if acc_split:
            out_copy(0)
            zero_acc()
        if pass_stride:
            load_table(1)
        plsc.subcore_barrier()
        main(gs1_hbm)
        plsc.subcore_barrier()
        out_copy(A if acc_split else 0)

    return pl.kernel(
        body,
        out_type=jax.ShapeDtypeStruct((_NC, out_rows, H), jnp.float32),
        mesh=mesh,
        compiler_params=pltpu.CompilerParams(use_tc_tiling_on_sc=False),
        scratch_types=[
            pltpu.VMEM_SHARED((A + 8, H), jnp.float32),
            pltpu.VMEM_SHARED((T2, H), jnp.float32),
            [pltpu.VMEM((_SG, 2, _CK), jnp.int32)] * _NI,
            [pltpu.VMEM((_CK, H), jnp.float32)] * _NB,
            [pltpu.SemaphoreType.DMA] * _NI,
            [pltpu.SemaphoreType.DMA] * _NB,
            [pltpu.SemaphoreType.DMA] * _NB,
        ],
    )


def _make_tc_scale(M, H, BN):
    def body(x, deg, out):
        out[...] = x[...] * deg[...]

    return pl.pallas_call(
        body,
        grid=(M // BN,),
        in_specs=[
            pl.BlockSpec((_NC, BN, H), lambda i: (0, i, 0)),
            pl.BlockSpec((1, BN, 1), lambda i: (0, i, 0)),
        ],
        out_specs=pl.BlockSpec((_NC, BN, H), lambda i: (0, i, 0)),
        out_shape=jax.ShapeDtypeStruct((_NC, M, H), jnp.float32),
    )


def _make_tc_final(N, D, BN):
    def body(xv2, x0, wt, degv, ab, out):
        a = ab[0, 0]
        b = ab[0, 1]
        xv = jnp.concatenate([xv2[0], xv2[1]], axis=1)
        xi = (1.0 - a) * (xv * degv[...]) + a * x0[...]
        mm = jnp.dot(xi, wt[...], preferred_element_type=jnp.float32)
        out[...] = (1.0 - b) * xi + b * mm

    H = D // 2
    return pl.pallas_call(
        body,
        grid=(N // BN,),
        in_specs=[
            pl.BlockSpec((_NC, BN, H), lambda i: (0, i, 0)),
            pl.BlockSpec((BN, D), lambda i: (i, 0)),
            pl.BlockSpec((D, D), lambda i: (0, 0)),
            pl.BlockSpec((BN, 1), lambda i: (i, 0)),
            pl.BlockSpec(memory_space=pltpu.SMEM),
        ],
        out_specs=pl.BlockSpec((BN, D), lambda i: (i, 0)),
        out_shape=jax.ShapeDtypeStruct((N, D), jnp.float32),
    )


def kernel(X, vertex, edges, X0, alpha, beta, W, degE, degV):
    N, D = X.shape
    E = vertex.shape[0]
    M = degE.shape[0]
    H = D // 2
    MH = M // 2

    xr = jnp.concatenate([X[:, :H], X[:, H:]], axis=0)
    zeros = jnp.zeros((_SR, H), jnp.float32)

    CH = -(-E // (_CK * _NS * _SG * _NI)) * _NS * _SG * _NI
    EP = CH * _CK

    def pad(a, val):
        return jnp.concatenate([a, jnp.full((EP - E,), val, jnp.int32)])

    def pack(g, sc):
        return jnp.stack([g.reshape(CH, _CK), sc.reshape(CH, _CK)], axis=1)

    vp = pad(vertex, 0)
    ep = pad(edges, M)
    spread = jnp.arange(EP, dtype=jnp.int32) % 8

    h1 = [pack(vp, jnp.where((ep >= p * MH) & (ep < (p + 1) * MH),
                             ep - p * MH, MH + spread)) for p in range(2)]
    xe2 = _make_sc_hop(N, MH, CH, H, N, 0, True, M)(h1[0], h1[1], xr, zeros)

    xe2 = _make_tc_scale(M, H, 1000)(xe2, degE.reshape(1, M, 1))

    h2 = []
    for p in range(2):
        ins = (ep >= p * MH) & (ep < (p + 1) * MH)
        h2.append(pack(jnp.where(ins, ep - p * MH, spread * 512),
                       jnp.where(ins, vp, N + spread)))
    xv2 = _make_sc_hop(MH, N, CH, H, M, MH, False, N)(
        h2[0], h2[1], xe2.reshape(2 * M, H), zeros)

    ab = jnp.stack([jnp.float32(alpha), jnp.float32(beta)]).reshape(1, 2)
    return _make_tc_final(N, D, 1000)(xv2, X0, W.T, degV, ab)


# speedup vs baseline: 1.9586x; 1.4529x over previous
import jax
import jax.numpy as jnp
from jax import lax
from jax.experimental import pallas as pl
from jax.experimental.pallas import tpu as pltpu
from jax.experimental.pallas import tpu_sc as plsc

_NC = 2
_NS = 16
_SR = 400

_NB = 4
_SG = 8
_NI = 4


def _split(total, parts, s):
    base, rem = divmod(total, parts)
    start = base * s + jnp.minimum(s, rem)
    cnt = jnp.where(s < rem, base + 1, base)
    return start, cnt


def _pipeline(gs_slice, ibs, rows, isem, gsem, ssem, tbl_hbm, acc_sh, NBODY):
    def iload(sg, p):
        pltpu.make_async_copy(gs_slice(sg * _SG), ibs[p], isem[p]).start()

    def iwait(p):
        pltpu.make_async_copy(gs_slice(0), ibs[p], isem[p]).wait()

    def gath(p, r, b):
        pltpu.make_async_copy(tbl_hbm.at[ibs[p].at[r, 0]], rows[b],
                              gsem[b]).start()

    def gwait(b):
        pltpu.make_async_copy(tbl_hbm.at[ibs[0].at[0, 0]], rows[b],
                              gsem[b]).wait()

    def scat(p, r, b):
        pltpu.make_async_copy(rows[b], acc_sh.at[ibs[p].at[r, 1]],
                              ssem[b]).start(add=True)

    def swait(b):
        pltpu.make_async_copy(rows[0], acc_sh.at[ibs[0].at[0, 1]],
                              ssem[b]).wait()

    iload(0, 0)
    iwait(0)
    for b in range(_NB):
        gath(0, b, b)

    def run_sg(g, u):
        p = u
        pn = (u + 1) % _NI
        sg = g * _NI + u
        last = (u == _NI - 1)

        def guarded(fn):
            if last:
                @pl.when(g < NBODY - 1)
                def _():
                    fn()
            else:
                fn()

        guarded(lambda: iload(sg + 1, pn))
        for k in range(_SG):
            b = k % _NB
            gwait(b)
            scat(p, k, b)
            if k == _NB:
                guarded(lambda: iwait(pn))
            if k < _SG - _NB:
                swait(b)
                gath(p, k + _NB, b)
            else:
                kk = k - (_SG - _NB)
                guarded(lambda bb=b, kk=kk: (swait(bb), gath(pn, kk, bb)))

    def group(g, carry):
        for u in range(_NI):
            run_sg(g, u)
        return carry

    lax.fori_loop(0, NBODY, group, 0)
    for b in range(_NB):
        swait(b)


def _make_sc_hop1(T, A, CH, H):
    CNT = CH // _NS
    NBODY = CNT // (_SG * _NI)
    mesh = plsc.VectorSubcoreMesh(core_axis_name="c", subcore_axis_name="s")

    def body(gs_hbm, tbl_hbm, z_hbm, out_hbm, acc_sh, ibs, rows,
             isem, gsem, ssem):
        c = lax.axis_index("c")
        s = lax.axis_index("s")

        za_start, za_cnt = _split(A // _SR, _NS, s)

        def zero_acc(k, carry):
            pltpu.sync_copy(z_hbm, acc_sh.at[pl.ds(k * _SR, _SR)])
            return carry

        lax.fori_loop(za_start, za_start + za_cnt, zero_acc, 0)
        plsc.subcore_barrier()

        def gs_slice(off):
            return gs_hbm.at[c, pl.ds(s * CNT + off, _SG)]

        _pipeline(gs_slice, ibs, rows, isem, gsem, ssem, tbl_hbm, acc_sh,
                  NBODY)
        plsc.subcore_barrier()

        def out_copy(k, carry):
            pltpu.sync_copy(acc_sh.at[pl.ds(k * _SR, _SR)],
                            out_hbm.at[c, pl.ds(k * _SR, _SR)])
            return carry

        lax.fori_loop(za_start, za_start + za_cnt, out_copy, 0)

    return pl.kernel(
        body,
        out_type=jax.ShapeDtypeStruct((_NC, A, H), jnp.float32),
        mesh=mesh,
        compiler_params=pltpu.CompilerParams(use_tc_tiling_on_sc=False),
        scratch_types=[
            pltpu.VMEM_SHARED((A + 8, H), jnp.float32),
            [pltpu.VMEM((_SG, 2, 128), jnp.int32)] * _NI,
            [pltpu.VMEM((128, H), jnp.float32)] * _NB,
            [pltpu.SemaphoreType.DMA] * _NI,
            [pltpu.SemaphoreType.DMA] * _NB,
            [pltpu.SemaphoreType.DMA] * _NB,
        ],
    )


def _make_sc_hop2(T, A, CH, D):
    CHC = CH // _NC
    CNT = CHC // _NS
    NBODY = CNT // (_SG * _NI)
    mesh = plsc.VectorSubcoreMesh(core_axis_name="c", subcore_axis_name="s")

    def body(gs_hbm, tbl_hbm, z_hbm, out_hbm, acc_sh, ibs, rows,
             isem, gsem, ssem):
        c = lax.axis_index("c")
        s = lax.axis_index("s")

        za_start, za_cnt = _split(A // _SR, _NS, s)

        def zero_acc(k, carry):
            pltpu.sync_copy(z_hbm, acc_sh.at[pl.ds(k * _SR, _SR)])
            return carry

        lax.fori_loop(za_start, za_start + za_cnt, zero_acc, 0)
        plsc.subcore_barrier()

        def gs_slice(off):
            return gs_hbm.at[pl.ds(c * CHC + s * CNT + off, _SG)]

        _pipeline(gs_slice, ibs, rows, isem, gsem, ssem, tbl_hbm, acc_sh,
                  NBODY)
        plsc.subcore_barrier()

        def out_copy(k, carry):
            pltpu.sync_copy(acc_sh.at[pl.ds(k * _SR, _SR)],
                            out_hbm.at[c, pl.ds(k * _SR, _SR)])
            return carry

        lax.fori_loop(za_start, za_start + za_cnt, out_copy, 0)

    return pl.kernel(
        body,
        out_type=jax.ShapeDtypeStruct((_NC, A, D), jnp.float32),
        mesh=mesh,
        compiler_params=pltpu.CompilerParams(use_tc_tiling_on_sc=False),
        scratch_types=[
            pltpu.VMEM_SHARED((A + 8, D), jnp.float32),
            [pltpu.VMEM((_SG, 2, 64), jnp.int32)] * _NI,
            [pltpu.VMEM((64, D), jnp.float32)] * _NB,
            [pltpu.SemaphoreType.DMA] * _NI,
            [pltpu.SemaphoreType.DMA] * _NB,
            [pltpu.SemaphoreType.DMA] * _NB,
        ],
    )


def _make_tc_scale(M, H, BN):
    def body(x, deg, out):
        out[...] = jnp.concatenate([x[0], x[1]], axis=1) * deg[...]

    return pl.pallas_call(
        body,
        grid=(M // BN,),
        in_specs=[
            pl.BlockSpec((_NC, BN, H), lambda i: (0, i, 0)),
            pl.BlockSpec((BN, 1), lambda i: (i, 0)),
        ],
        out_specs=pl.BlockSpec((BN, 2 * H), lambda i: (i, 0)),
        out_shape=jax.ShapeDtypeStruct((M, 2 * H), jnp.float32),
    )


def _make_tc_final(N, D, BN):
    def body(xv2, x0, wt, degv, ab, out):
        a = ab[0, 0]
        b = ab[0, 1]
        xv = xv2[0] + xv2[1]
        xi = (1.0 - a) * (xv * degv[...]) + a * x0[...]
        mm = jnp.dot(xi, wt[...], preferred_element_type=jnp.float32)
        out[...] = (1.0 - b) * xi + b * mm

    return pl.pallas_call(
        body,
        grid=(N // BN,),
        in_specs=[
            pl.BlockSpec((_NC, BN, D), lambda i: (0, i, 0)),
            pl.BlockSpec((BN, D), lambda i: (i, 0)),
            pl.BlockSpec((D, D), lambda i: (0, 0)),
            pl.BlockSpec((BN, 1), lambda i: (i, 0)),
            pl.BlockSpec(memory_space=pltpu.SMEM),
        ],
        out_specs=pl.BlockSpec((BN, D), lambda i: (i, 0)),
        out_shape=jax.ShapeDtypeStruct((N, D), jnp.float32),
    )


def kernel(X, vertex, edges, X0, alpha, beta, W, degE, degV):
    N, D = X.shape
    E = vertex.shape[0]
    M = degE.shape[0]
    H = D // 2

    xr = jnp.concatenate([X[:, :H], X[:, H:]], axis=0)
    zeros_h = jnp.zeros((_SR, H), jnp.float32)
    zeros_d = jnp.zeros((_SR, D), jnp.float32)

    CH = -(-E // (128 * _NS * _SG * _NI)) * _NS * _SG * _NI
    EP = CH * 128
    CH2 = EP // 64

    def pad(a, val):
        return jnp.concatenate([a, jnp.full((EP - E,), val, jnp.int32)])

    valid = jnp.arange(EP) < E
    spread = jnp.arange(EP, dtype=jnp.int32) % 8

    vp = pad(vertex, 0)
    g1 = jnp.stack([vp, vp + N]).reshape(_NC, CH, 128)
    s1 = jnp.broadcast_to(
        jnp.where(valid, pad(edges, 0),
                  M + spread).reshape(1, CH, 128), (_NC, CH, 128))
    gs1 = jnp.stack([g1, s1], axis=2)
    xe2 = _make_sc_hop1(2 * N, M, CH, H)(gs1, xr, zeros_h)

    xe = _make_tc_scale(M, H, 1000)(xe2, degE)

    g2 = jnp.where(valid, pad(edges, 0), spread * 512)
    s2 = jnp.where(valid, pad(vertex, 0), N + spread)
    gs2 = jnp.stack([g2.reshape(CH2, 64), s2.reshape(CH2, 64)], axis=1)
    xv2 = _make_sc_hop2(M, N, CH2, D)(gs2, xe, zeros_d)

    ab = jnp.stack([jnp.float32(alpha), jnp.float32(beta)]).reshape(1, 2)
    return _make_tc_final(N, D, 1000)(xv2, X0, W.T, degV, ab)
